# Initial kernel scaffold; baseline (speedup 1.0000x reference)
import functools

import jax
import jax.numpy as jnp
from jax import lax
from jax.experimental import pallas as pl
from jax.experimental.pallas import tpu as pltpu
from jax.experimental.pallas import tpu_sc as plsc

ROWS, COLS = 16384, 128
NUM_CORES, NUM_SUBCORES = 2, 16
NUM_WORKERS = NUM_CORES * NUM_SUBCORES
ROWS_PER_WORKER = ROWS // NUM_WORKERS
LANES = 16
NSL = COLS // LANES
BLK = 64

_MESH = plsc.VectorSubcoreMesh(
    core_axis_name="c", subcore_axis_name="s",
    num_cores=NUM_CORES, num_subcores=NUM_SUBCORES,
)


def _reduce_body(x_hbm, part_hbm, rows_v, acc_v, comb_v, shared):
    cid = lax.axis_index("c")
    sid = lax.axis_index("s")
    wid = cid * NUM_SUBCORES + sid
    base = wid * ROWS_PER_WORKER

    pltpu.sync_copy(x_hbm.at[pl.ds(base, ROWS_PER_WORKER)], rows_v)

    def body(r, accs):
        return tuple(
            a + rows_v[r, pl.ds(LANES * j, LANES)] for j, a in enumerate(accs)
        )

    accs = lax.fori_loop(
        0, ROWS_PER_WORKER, body,
        tuple(jnp.zeros((LANES,), jnp.float32) for _ in range(NSL)),
    )
    for j, a in enumerate(accs):
        acc_v[pl.ds(LANES * j, LANES)] = a

    pltpu.sync_copy(acc_v, shared.at[sid])
    plsc.subcore_barrier()

    @pl.when(sid == 0)
    def _():
        pltpu.sync_copy(shared, comb_v)
        for j in range(NSL):
            t = comb_v[0, pl.ds(LANES * j, LANES)]
            for rr in range(1, NUM_SUBCORES):
                t = t + comb_v[rr, pl.ds(LANES * j, LANES)]
            acc_v[pl.ds(LANES * j, LANES)] = t
        pltpu.sync_copy(acc_v, part_hbm.at[cid])


def _write_body(part_hbm, out_hbm, part_v, block_v):
    cid = lax.axis_index("c")
    sid = lax.axis_index("s")
    wid = cid * NUM_SUBCORES + sid
    base = wid * ROWS_PER_WORKER

    pltpu.sync_copy(part_hbm, part_v)

    iota = lax.broadcasted_iota(jnp.int32, (LANES,), 0)
    mins = []
    for j in range(NSL):
        s_j = part_v[0, pl.ds(LANES * j, LANES)] + part_v[1, pl.ds(LANES * j, LANES)]
        cand = jnp.where(s_j >= 0.0, iota + LANES * j, COLS)
        mins.append(jnp.min(cand))
    idx = functools.reduce(jnp.minimum, mins)
    idx = jnp.where(idx >= COLS, 0, idx)

    vrow = [
        jnp.where(iota + LANES * j == idx, 1.0, -1.0).astype(jnp.float32)
        for j in range(NSL)
    ]

    def fill(r, carry):
        for j in range(NSL):
            block_v[r, pl.ds(LANES * j, LANES)] = vrow[j]
        return carry

    lax.fori_loop(0, BLK, fill, 0)

    for b in range(ROWS_PER_WORKER // BLK):
        pltpu.sync_copy(block_v, out_hbm.at[pl.ds(base + b * BLK, BLK)])


_reduce = pl.kernel(
    _reduce_body,
    out_type=jax.ShapeDtypeStruct((NUM_CORES, COLS), jnp.float32),
    mesh=_MESH,
    scratch_types=[
        pltpu.VMEM((ROWS_PER_WORKER, COLS), jnp.float32),
        pltpu.VMEM((COLS,), jnp.float32),
        pltpu.VMEM((NUM_SUBCORES, COLS), jnp.float32),
        pltpu.VMEM_SHARED((NUM_SUBCORES, COLS), jnp.float32),
    ],
)

_write = pl.kernel(
    _write_body,
    out_type=jax.ShapeDtypeStruct((ROWS, COLS), jnp.float32),
    mesh=_MESH,
    scratch_types=[
        pltpu.VMEM((NUM_CORES, COLS), jnp.float32),
        pltpu.VMEM((BLK, COLS), jnp.float32),
    ],
)


@jax.jit
def kernel(x):
    return _write(_reduce(x))


# speedup vs baseline: 2.5951x; 2.5951x over previous
import functools

import jax
import jax.numpy as jnp
from jax import lax
from jax.experimental import pallas as pl
from jax.experimental.pallas import tpu as pltpu
from jax.experimental.pallas import tpu_sc as plsc

ROWS, COLS = 16384, 128
NUM_CORES, NUM_SUBCORES = 2, 16
NUM_WORKERS = NUM_CORES * NUM_SUBCORES
ROWS_PER_WORKER = ROWS // NUM_WORKERS
LANES = 16
NSL = COLS // LANES
BLK = 64

_MESH = plsc.VectorSubcoreMesh(
    core_axis_name="c", subcore_axis_name="s",
    num_cores=NUM_CORES, num_subcores=NUM_SUBCORES,
)


CHUNK = 64
NCHUNK = ROWS_PER_WORKER // CHUNK


def _reduce_body(x_hbm, part_hbm, rows_v, acc_v, comb_v, shared, sem0, sem1):
    cid = lax.axis_index("c")
    sid = lax.axis_index("s")
    wid = cid * NUM_SUBCORES + sid
    base = wid * ROWS_PER_WORKER

    sems = (sem0, sem1)
    copies = [None, None]
    copies[0] = pltpu.make_async_copy(
        x_hbm.at[pl.ds(base, CHUNK)], rows_v.at[0], sems[0]
    )
    copies[0].start()

    accs = tuple(jnp.zeros((LANES,), jnp.float32) for _ in range(NSL))
    for k in range(NCHUNK):
        buf = k % 2
        if k + 1 < NCHUNK:
            copies[1 - buf] = pltpu.make_async_copy(
                x_hbm.at[pl.ds(base + (k + 1) * CHUNK, CHUNK)],
                rows_v.at[1 - buf],
                sems[1 - buf],
            )
            copies[1 - buf].start()
        copies[buf].wait()

        def body(r, accs, buf=buf):
            return tuple(
                a + rows_v[buf, r, pl.ds(LANES * j, LANES)]
                for j, a in enumerate(accs)
            )

        accs = lax.fori_loop(0, CHUNK, body, accs)

    for j, a in enumerate(accs):
        acc_v[pl.ds(LANES * j, LANES)] = a

    pltpu.sync_copy(acc_v, shared.at[sid])
    plsc.subcore_barrier()

    @pl.when(sid == 0)
    def _():
        pltpu.sync_copy(shared, comb_v)
        for j in range(NSL):
            t = comb_v[0, pl.ds(LANES * j, LANES)]
            for rr in range(1, NUM_SUBCORES):
                t = t + comb_v[rr, pl.ds(LANES * j, LANES)]
            acc_v[pl.ds(LANES * j, LANES)] = t
        pltpu.sync_copy(acc_v, part_hbm.at[cid])


def _write_body(part_hbm, out_hbm, part_v, block_v, out_sem):
    cid = lax.axis_index("c")
    sid = lax.axis_index("s")
    wid = cid * NUM_SUBCORES + sid
    base = wid * ROWS_PER_WORKER

    pltpu.sync_copy(part_hbm, part_v)

    iota = lax.broadcasted_iota(jnp.int32, (LANES,), 0)
    cands = []
    for j in range(NSL):
        s_j = part_v[0, pl.ds(LANES * j, LANES)] + part_v[1, pl.ds(LANES * j, LANES)]
        ffs = plsc.all_reduce_ffs(s_j >= 0.0)
        cands.append(jnp.where(ffs < LANES, ffs + LANES * j, COLS))
    idx = functools.reduce(jnp.minimum, cands)
    idx = jnp.where(idx >= COLS, 0, idx)

    vrow = [
        jnp.where(iota + LANES * j == idx, 1.0, -1.0).astype(jnp.float32)
        for j in range(NSL)
    ]

    def fill(r, carry):
        for j in range(NSL):
            block_v[r, pl.ds(LANES * j, LANES)] = vrow[j]
        return carry

    lax.fori_loop(0, BLK, fill, 0)

    copies = [
        pltpu.make_async_copy(
            block_v, out_hbm.at[pl.ds(base + b * BLK, BLK)], out_sem
        )
        for b in range(ROWS_PER_WORKER // BLK)
    ]
    for c in copies:
        c.start()
    for c in copies:
        c.wait()


_PARAMS = pltpu.CompilerParams(needs_layout_passes=False)

_reduce = pl.kernel(
    _reduce_body,
    out_type=jax.ShapeDtypeStruct((NUM_CORES, COLS), jnp.float32),
    mesh=_MESH,
    compiler_params=_PARAMS,
    scratch_types=[
        pltpu.VMEM((2, CHUNK, COLS), jnp.float32),
        pltpu.VMEM((COLS,), jnp.float32),
        pltpu.VMEM((NUM_SUBCORES, COLS), jnp.float32),
        pltpu.VMEM_SHARED((NUM_SUBCORES, COLS), jnp.float32),
        pltpu.SemaphoreType.DMA,
        pltpu.SemaphoreType.DMA,
    ],
)

_write = pl.kernel(
    _write_body,
    out_type=jax.ShapeDtypeStruct((ROWS, COLS), jnp.float32),
    mesh=_MESH,
    compiler_params=_PARAMS,
    scratch_types=[
        pltpu.VMEM((NUM_CORES, COLS), jnp.float32),
        pltpu.VMEM((BLK, COLS), jnp.float32),
        pltpu.SemaphoreType.DMA,
    ],
)


@jax.jit
def kernel(x):
    return _write(_reduce(x))
